# baseline (speedup 1.0000x reference)
import functools

import jax
import jax.numpy as jnp
from jax import lax
from jax.experimental import pallas as pl
from jax.experimental.pallas import tpu as pltpu
from jax.experimental.pallas import tpu_sc as plsc

VOCAB = 50304
DIM = 768
NEMB = 6
BATCH = 4
SEQ = 2048

NW = 32
ROWS = BATCH * SEQ
RPW = ROWS // NW
CHUNK = 64
NCHUNK = RPW // CHUNK
NBUF = 2

_mesh = plsc.VectorSubcoreMesh(core_axis_name="c", subcore_axis_name="s")


@functools.partial(
    pl.kernel,
    mesh=_mesh,
    out_type=jax.ShapeDtypeStruct((ROWS, DIM), jnp.float32),
    scratch_types=(
        [pltpu.VMEM((RPW,), jnp.int32)]
        + [pltpu.VMEM((CHUNK, DIM), jnp.float32)] * NBUF
        + [pltpu.SemaphoreType.DMA] * (2 * NBUF)
    ),
)
def _gather1(idx_hbm, tab_hbm, out, idx_v, *rest):
    bufs = rest[:NBUF]
    gsems = rest[NBUF:2 * NBUF]
    wsems = rest[2 * NBUF:]
    wid = lax.axis_index("s") * 2 + lax.axis_index("c")
    base = wid * RPW
    pltpu.sync_copy(idx_hbm.at[wid], idx_v)

    def gather(step):
        b = step % NBUF
        return pltpu.async_copy(
            tab_hbm.at[idx_v.at[pl.ds(step * CHUNK, CHUNK)]],
            bufs[b], gsems[b])

    def write(step):
        b = step % NBUF
        return pltpu.async_copy(
            bufs[b], out.at[pl.ds(base + step * CHUNK, CHUNK)], wsems[b])

    writes = [None] * NCHUNK
    gathers = [None] * NCHUNK
    for s in range(min(NBUF - 1, NCHUNK)):
        gathers[s] = gather(s)
    for s in range(NCHUNK):
        gathers[s].wait()
        writes[s] = write(s)
        nxt = s + NBUF - 1
        if nxt < NCHUNK:
            if s >= 1:
                writes[s - 1].wait()
            gathers[nxt] = gather(nxt)
    for s in range(max(0, NCHUNK - NBUF), NCHUNK):
        writes[s].wait()


def kernel(inputs, tables):
    flat = inputs.reshape(-1).astype(jnp.int32)
    offs = (jnp.arange(NEMB, dtype=jnp.int32) * VOCAB)[:, None]
    idx_all = (flat[None, :] + offs).reshape(NEMB, NW, RPW)
    tab = tables.reshape(NEMB * VOCAB, DIM)
    ve = [_gather1(idx_all[t], tab).reshape(BATCH, SEQ, DIM)
          for t in range(NEMB)]
    return tuple(ve + ve[::-1])
